# Initial kernel scaffold; baseline (speedup 1.0000x reference)
import jax
import jax.numpy as jnp
from jax.experimental import pallas as pl


def kernel(hidden_states, W_gate, gate_w, up_w, down_w, shared_gate_w, shared_up_w, shared_down_w):
    raise NotImplementedError("write your pallas kernel here")


# speedup vs baseline: 1.2042x; 1.2042x over previous
import functools

import jax
import jax.numpy as jnp
from jax.experimental import pallas as pl
from jax.experimental.pallas import tpu as pltpu

_E = 8
_I = 512
_TB = 256


def _silu(x):
    return x * jax.nn.sigmoid(x)


def _dot(a, b, dims, prec):
    return jax.lax.dot_general(a, b, (dims, ((), ())),
                               preferred_element_type=jnp.float32,
                               precision=prec)


def _moe_body(x_ref, wg_ref, gw_ref, uw_ref, dw_ref, sgw_ref, suw_ref,
              sdw_ref, out_ref, c_ref):
    e = pl.program_id(1)
    prec = jax.lax.Precision.DEFAULT
    x = x_ref[...]

    @pl.when(e == 0)
    def _router():
        logits = _dot(x, wg_ref[...], ((1,), (1,)), prec)
        m = jnp.max(logits, axis=1, keepdims=True)
        p = jnp.exp(logits - m)
        iota = jax.lax.broadcasted_iota(jnp.int32, p.shape, 1)
        m1 = jnp.max(p, axis=1, keepdims=True)
        i1 = jnp.min(jnp.where(p == m1, iota, _E), axis=1, keepdims=True)
        p2 = jnp.where(iota == i1, -1.0, p)
        m2 = jnp.max(p2, axis=1, keepdims=True)
        i2 = jnp.min(jnp.where(p2 == m2, iota, _E), axis=1, keepdims=True)
        c = jnp.where(iota == i1, m1, jnp.where(iota == i2, m2, 0.0))
        c_ref[...] = c / (m1 + m2 + 1e-20)
        out_ref[...] = jnp.zeros_like(out_ref)

    sg = _dot(x, sgw_ref[...], ((1,), (1,)), prec)
    su = _dot(x, suw_ref[...], ((1,), (1,)), prec)
    sa = _silu(sg) * su
    shared = _dot(sa, sdw_ref[...], ((1,), (1,)), prec)

    gw = gw_ref[0]
    uw = uw_ref[0]
    dw = dw_ref[0]
    g = _dot(x, gw, ((1,), (1,)), prec)
    u = _dot(x, uw, ((1,), (1,)), prec)
    a = _silu(g) * u
    h = _dot(a, dw, ((1,), (1,)), prec)

    iota = jax.lax.broadcasted_iota(jnp.int32, c_ref.shape, 1)
    w_col = jnp.sum(jnp.where(iota == e, c_ref[...], 0.0), axis=1)
    out_ref[...] += shared + w_col[:, None] * h


def kernel(hidden_states, W_gate, gate_w, up_w, down_w,
           shared_gate_w, shared_up_w, shared_down_w):
    bsz, seq, hdim = hidden_states.shape
    T = bsz * seq
    SI = shared_gate_w.shape[0]
    sc = SI // _E
    x = hidden_states.reshape(T, hdim)

    out = pl.pallas_call(
        _moe_body,
        grid=(T // _TB, _E),
        in_specs=[
            pl.BlockSpec((_TB, hdim), lambda t, e: (t, 0)),
            pl.BlockSpec((_E, hdim), lambda t, e: (0, 0)),
            pl.BlockSpec((1, _I, hdim), lambda t, e: (e, 0, 0)),
            pl.BlockSpec((1, _I, hdim), lambda t, e: (e, 0, 0)),
            pl.BlockSpec((1, hdim, _I), lambda t, e: (e, 0, 0)),
            pl.BlockSpec((sc, hdim), lambda t, e: (e, 0)),
            pl.BlockSpec((sc, hdim), lambda t, e: (e, 0)),
            pl.BlockSpec((hdim, sc), lambda t, e: (0, e)),
        ],
        out_specs=pl.BlockSpec((_TB, hdim), lambda t, e: (t, 0)),
        out_shape=jax.ShapeDtypeStruct((T, hdim), jnp.float32),
        scratch_shapes=[pltpu.VMEM((_TB, _E), jnp.float32)],
        compiler_params=pltpu.CompilerParams(
            dimension_semantics=("parallel", "arbitrary")),
    )(x, W_gate, gate_w, up_w, down_w, shared_gate_w, shared_up_w,
      shared_down_w)
    return out.reshape(bsz, seq, hdim)
